# baseline (speedup 1.0000x reference)
import functools

import numpy as np
import jax
import jax.numpy as jnp
from jax import lax
from jax.experimental import pallas as pl
from jax.experimental.pallas import tpu as pltpu
from jax.experimental.pallas import tpu_sc as plsc

_NUM_FIELDS = 26
_FIELD_DIM = 100000
_EMBED_DIM = 16
_BATCH = 16384
_B = _BATCH * _NUM_FIELDS

_NW = 32
_BPW = _B // _NW
_CHUNK = 1664
_NCHUNK = _BPW // _CHUNK
_NSUB = _CHUNK // 128
_NVEC = _CHUNK // 16

_OFFSETS = np.arange(_NUM_FIELDS, dtype=np.int32) * _FIELD_DIM
_OFF_TILE = np.tile(_OFFSETS, _CHUNK // _NUM_FIELDS)


@functools.partial(
    pl.kernel,
    out_type=jax.ShapeDtypeStruct((_B, _EMBED_DIM), jnp.float32),
    mesh=plsc.VectorSubcoreMesh(core_axis_name="c", subcore_axis_name="s"),
    compiler_params=pltpu.CompilerParams(use_tc_tiling_on_sc=False),
    scratch_types=[
        pltpu.VMEM((_CHUNK,), jnp.int32),
        pltpu.VMEM((_CHUNK,), jnp.int32),
        pltpu.VMEM((_CHUNK,), jnp.int32),
        pltpu.VMEM((_CHUNK, _EMBED_DIM), jnp.float32),
        pltpu.SemaphoreType.DMA,
    ],
)
def _emb_lookup(x_hbm, off_hbm, table_hbm, out_hbm, x_v, idx_v, off_v, rows_v, sem):
    wid = lax.axis_index("s") * 2 + lax.axis_index("c")
    base = wid * _BPW
    pltpu.sync_copy(off_hbm, off_v)

    def do_chunk(ci, carry):
        start = base + ci * _CHUNK
        pltpu.sync_copy(x_hbm.at[pl.ds(start, _CHUNK)], x_v)

        def compute(vi, c2):
            p = pl.ds(vi * 16, 16)
            xv = x_v[p]
            iv = jnp.where(xv == 0, 0, xv + off_v[p])
            idx_v[p] = iv
            return c2

        lax.fori_loop(0, _NVEC, compute, 0)

        copies = [
            pltpu.async_copy(
                table_hbm.at[idx_v.at[pl.ds(j * 128, 128)]],
                rows_v.at[pl.ds(j * 128, 128)],
                sem,
            )
            for j in range(_NSUB)
        ]
        for cp in copies:
            cp.wait()
        pltpu.sync_copy(rows_v, out_hbm.at[pl.ds(start, _CHUNK)])
        return carry

    lax.fori_loop(0, _NCHUNK, do_chunk, 0)


def kernel(x, table):
    out = _emb_lookup(x.reshape(-1), jnp.asarray(_OFF_TILE), table)
    return out.reshape(_BATCH, _NUM_FIELDS, _EMBED_DIM)


# speedup vs baseline: 1.2068x; 1.2068x over previous
import functools

import jax
import jax.numpy as jnp
from jax import lax
from jax.experimental import pallas as pl
from jax.experimental.pallas import tpu as pltpu
from jax.experimental.pallas import tpu_sc as plsc

_NUM_FIELDS = 26
_FIELD_DIM = 100000
_EMBED_DIM = 16
_BATCH = 16384
_NW = 32
_BPW = _BATCH // _NW
_NVEC = _BPW // 16


@functools.partial(
    pl.kernel,
    out_type=jax.ShapeDtypeStruct((_NUM_FIELDS, _EMBED_DIM, _BATCH), jnp.float32),
    mesh=plsc.VectorSubcoreMesh(core_axis_name="c", subcore_axis_name="s"),
    compiler_params=pltpu.CompilerParams(
        use_tc_tiling_on_sc=False, needs_layout_passes=False
    ),
    scratch_types=[
        pltpu.VMEM((_BPW,), jnp.int32),
        pltpu.VMEM((_BPW,), jnp.int32),
        pltpu.VMEM((_BPW, _EMBED_DIM), jnp.float32),
        pltpu.VMEM((_EMBED_DIM, _BPW), jnp.float32),
        pltpu.VMEM((16,), jnp.int32),
        pltpu.SemaphoreType.DMA,
    ],
)
def _emb_lookup(xt_hbm, table_hbm, out_hbm, x_v, idx_v, rows_v, valt_v, iota_v, sem):
    wid = lax.axis_index("s") * 2 + lax.axis_index("c")
    b0 = wid * _BPW
    iota_v[...] = lax.iota(jnp.int32, 16)

    def do_field(f, carry):
        pltpu.sync_copy(xt_hbm.at[f].at[pl.ds(b0, _BPW)], x_v)
        off = f * _FIELD_DIM

        def compute(vi, c2):
            p = pl.ds(vi * 16, 16)
            xv = x_v[p]
            idx_v[p] = jnp.where(xv == 0, 0, xv + off)
            return c2

        lax.fori_loop(0, _NVEC, compute, 0)

        copies = [
            pltpu.async_copy(
                table_hbm.at[idx_v.at[pl.ds(j * 128, 128)]],
                rows_v.at[pl.ds(j * 128, 128)],
                sem,
            )
            for j in range(_BPW // 128)
        ]
        for cp in copies:
            cp.wait()

        def transpose16(g, c2):
            ii = iota_v[...]
            for r16 in range(16):
                r = g * 16 + r16
                rv = jnp.full((16,), r, jnp.int32)
                plsc.store_scatter(valt_v, [ii, rv], rows_v[r, :])
            return c2

        lax.fori_loop(0, _NVEC, transpose16, 0)
        pltpu.sync_copy(valt_v, out_hbm.at[f].at[:, pl.ds(b0, _BPW)])
        return carry

    lax.fori_loop(0, _NUM_FIELDS, do_field, 0)


def kernel(x, table):
    out = _emb_lookup(x.T, table)
    return jnp.transpose(out, (2, 0, 1))


# speedup vs baseline: 1.2092x; 1.0020x over previous
import functools

import jax
import jax.numpy as jnp
from jax import lax
from jax.experimental import pallas as pl
from jax.experimental.pallas import tpu as pltpu
from jax.experimental.pallas import tpu_sc as plsc

_NUM_FIELDS = 26
_FIELD_DIM = 100000
_EMBED_DIM = 16
_BATCH = 16384
_NW = 32
_BPW = _BATCH // _NW
_NVEC = _BPW // 16


@functools.partial(
    pl.kernel,
    out_type=jax.ShapeDtypeStruct((_NUM_FIELDS * _BATCH,), jnp.int32),
    mesh=plsc.VectorSubcoreMesh(core_axis_name="c", subcore_axis_name="s"),
    compiler_params=pltpu.CompilerParams(
        use_tc_tiling_on_sc=True, needs_layout_passes=False
    ),
    scratch_types=[
        pltpu.VMEM((_BPW,), jnp.int32),
        pltpu.VMEM((_BPW,), jnp.int32),
    ],
)
def _make_indices(xt_hbm, idx_hbm, x_v, idx_v):
    wid = lax.axis_index("s") * 2 + lax.axis_index("c")
    b0 = wid * _BPW

    def do_field(f, carry):
        pltpu.sync_copy(xt_hbm.at[f].at[pl.ds(b0, _BPW)], x_v)
        off = f * _FIELD_DIM

        def compute(vi, c2):
            p = pl.ds(vi * 16, 16)
            xv = x_v[p]
            idx_v[p] = jnp.where(xv == 0, 0, xv + off)
            return c2

        lax.fori_loop(0, _NVEC, compute, 0)
        pltpu.sync_copy(idx_v, idx_hbm.at[pl.ds(f * _BATCH + b0, _BPW)])
        return carry

    lax.fori_loop(0, _NUM_FIELDS, do_field, 0)


@functools.partial(
    pl.kernel,
    out_type=jax.ShapeDtypeStruct((_NUM_FIELDS, _EMBED_DIM, _BATCH), jnp.float32),
    mesh=plsc.VectorSubcoreMesh(core_axis_name="c", subcore_axis_name="s"),
    compiler_params=pltpu.CompilerParams(
        use_tc_tiling_on_sc=False, needs_layout_passes=False
    ),
    scratch_types=[
        pltpu.VMEM((_BPW,), jnp.int32),
        pltpu.VMEM((_BPW, _EMBED_DIM), jnp.float32),
        pltpu.VMEM((_EMBED_DIM, _BPW), jnp.float32),
        pltpu.VMEM((16,), jnp.int32),
        pltpu.SemaphoreType.DMA,
    ],
)
def _emb_lookup(idx_hbm, table_hbm, out_hbm, idx_v, rows_v, valt_v, iota_v, sem):
    wid = lax.axis_index("s") * 2 + lax.axis_index("c")
    b0 = wid * _BPW
    iota_v[...] = lax.iota(jnp.int32, 16)

    def do_field(f, carry):
        pltpu.sync_copy(idx_hbm.at[pl.ds(f * _BATCH + b0, _BPW)], idx_v)

        copies = [
            pltpu.async_copy(
                table_hbm.at[idx_v.at[pl.ds(j * 128, 128)]],
                rows_v.at[pl.ds(j * 128, 128)],
                sem,
            )
            for j in range(_BPW // 128)
        ]
        for cp in copies:
            cp.wait()

        def transpose16(g, c2):
            ii = iota_v[...]
            for r16 in range(16):
                r = g * 16 + r16
                rv = jnp.full((16,), r, jnp.int32)
                plsc.store_scatter(valt_v, [ii, rv], rows_v[r, :])
            return c2

        lax.fori_loop(0, _NVEC, transpose16, 0)
        pltpu.sync_copy(valt_v, out_hbm.at[f].at[:, pl.ds(b0, _BPW)])
        return carry

    lax.fori_loop(0, _NUM_FIELDS, do_field, 0)


def kernel(x, table):
    idx = _make_indices(x.T)
    out = _emb_lookup(idx, table)
    return jnp.transpose(out, (2, 0, 1))


# speedup vs baseline: 2.3025x; 1.9042x over previous
import functools

import jax
import jax.numpy as jnp
from jax import lax
from jax.experimental import pallas as pl
from jax.experimental.pallas import tpu as pltpu
from jax.experimental.pallas import tpu_sc as plsc

_NUM_FIELDS = 26
_FIELD_DIM = 100000
_EMBED_DIM = 16
_BATCH = 16384
_NW = 32
_BPW = _BATCH // _NW
_NVEC = _BPW // 16

_NROW = 2600001
_R = 1024
_NFULL = _NROW // _R
_TAILROW = _NFULL * _R
_TAIL = _NROW - _TAILROW
_CPW_HI = _NFULL - (_NFULL // _NW) * _NW


@functools.partial(
    pl.kernel,
    out_type=jax.ShapeDtypeStruct((_NROW * 16,), jnp.float32),
    mesh=plsc.VectorSubcoreMesh(core_axis_name="c", subcore_axis_name="s"),
    compiler_params=pltpu.CompilerParams(
        use_tc_tiling_on_sc=True, needs_layout_passes=False
    ),
    scratch_types=[
        pltpu.VMEM((16, _R), jnp.float32),
        pltpu.VMEM((_R * 16,), jnp.float32),
        pltpu.VMEM((16, _TAIL), jnp.float32),
        pltpu.VMEM((16,), jnp.int32),
    ],
)
def _linearize_table(tt_hbm, tlin_hbm, in_v, out_v, tin_v, iota_v):
    wid = lax.axis_index("s") * 2 + lax.axis_index("c")
    iota_v[...] = lax.iota(jnp.int32, 16) * 16
    ii16 = iota_v[...]

    nch = jnp.where(wid < _CPW_HI, _NFULL // _NW + 1, _NFULL // _NW)

    def do_chunk(k, carry):
        cid = wid + _NW * k
        r0 = cid * _R
        pltpu.sync_copy(tt_hbm.at[:, pl.ds(r0, _R)], in_v)

        def tr_vec(g, c2):
            for c in range(16):
                plsc.store_scatter(out_v, [ii16 + (g * 256 + c)],
                                   in_v[c, pl.ds(g * 16, 16)])
            return c2

        lax.fori_loop(0, _R // 16, tr_vec, 0)
        pltpu.sync_copy(out_v, tlin_hbm.at[pl.ds(r0 * 16, _R * 16)])
        return carry

    lax.fori_loop(0, nch, do_chunk, 0)

    @pl.when(wid == _NW - 1)
    def _tail():
        pltpu.sync_copy(tt_hbm.at[:, pl.ds(_TAILROW, _TAIL)], tin_v)
        starts = [g * 16 for g in range(_TAIL // 16)] + [_TAIL - 16]
        for s in starts:
            for c in range(16):
                plsc.store_scatter(out_v, [ii16 + (s * 16 + c)],
                                   tin_v[c, pl.ds(s, 16)])
        pltpu.sync_copy(out_v.at[pl.ds(0, _TAIL * 16)],
                        tlin_hbm.at[pl.ds(_TAILROW * 16, _TAIL * 16)])


@functools.partial(
    pl.kernel,
    out_type=jax.ShapeDtypeStruct((_NUM_FIELDS * _BATCH,), jnp.int32),
    mesh=plsc.VectorSubcoreMesh(core_axis_name="c", subcore_axis_name="s"),
    compiler_params=pltpu.CompilerParams(
        use_tc_tiling_on_sc=True, needs_layout_passes=False
    ),
    scratch_types=[
        pltpu.VMEM((_BPW,), jnp.int32),
        pltpu.VMEM((_BPW,), jnp.int32),
    ],
)
def _make_indices(xt_hbm, idx_hbm, x_v, idx_v):
    wid = lax.axis_index("s") * 2 + lax.axis_index("c")
    b0 = wid * _BPW

    def do_field(f, carry):
        pltpu.sync_copy(xt_hbm.at[f].at[pl.ds(b0, _BPW)], x_v)
        off = f * _FIELD_DIM

        def compute(vi, c2):
            p = pl.ds(vi * 16, 16)
            xv = x_v[p]
            idx_v[p] = jnp.where(xv == 0, 0, xv + off)
            return c2

        lax.fori_loop(0, _NVEC, compute, 0)
        pltpu.sync_copy(idx_v, idx_hbm.at[pl.ds(f * _BATCH + b0, _BPW)])
        return carry

    lax.fori_loop(0, _NUM_FIELDS, do_field, 0)


@functools.partial(
    pl.kernel,
    out_type=jax.ShapeDtypeStruct((_NUM_FIELDS, _EMBED_DIM, _BATCH), jnp.float32),
    mesh=plsc.VectorSubcoreMesh(core_axis_name="c", subcore_axis_name="s"),
    compiler_params=pltpu.CompilerParams(
        use_tc_tiling_on_sc=False, needs_layout_passes=False
    ),
    scratch_types=[
        pltpu.VMEM((_BPW,), jnp.int32),
        pltpu.VMEM((_BPW, _EMBED_DIM), jnp.float32),
        pltpu.VMEM((_EMBED_DIM, _BPW), jnp.float32),
        pltpu.VMEM((16,), jnp.int32),
        pltpu.SemaphoreType.DMA,
    ],
)
def _emb_lookup(idx_hbm, table_hbm, out_hbm, idx_v, rows_v, valt_v, iota_v, sem):
    wid = lax.axis_index("s") * 2 + lax.axis_index("c")
    b0 = wid * _BPW
    iota_v[...] = lax.iota(jnp.int32, 16)

    def do_field(f, carry):
        pltpu.sync_copy(idx_hbm.at[pl.ds(f * _BATCH + b0, _BPW)], idx_v)

        copies = [
            pltpu.async_copy(
                table_hbm.at[idx_v.at[pl.ds(j * 128, 128)]],
                rows_v.at[pl.ds(j * 128, 128)],
                sem,
            )
            for j in range(_BPW // 128)
        ]
        for cp in copies:
            cp.wait()

        def transpose16(g, c2):
            ii = iota_v[...]
            for r16 in range(16):
                r = g * 16 + r16
                rv = jnp.full((16,), r, jnp.int32)
                plsc.store_scatter(valt_v, [ii, rv], rows_v[r, :])
            return c2

        lax.fori_loop(0, _NVEC, transpose16, 0)
        pltpu.sync_copy(valt_v, out_hbm.at[f].at[:, pl.ds(b0, _BPW)])
        return carry

    lax.fori_loop(0, _NUM_FIELDS, do_field, 0)


def kernel(x, table):
    tlin = _linearize_table(table.T)
    t2d = tlin.reshape(_NROW, _EMBED_DIM)
    idx = _make_indices(x.T)
    out = _emb_lookup(idx, t2d)
    return jnp.transpose(out, (2, 0, 1))


# speedup vs baseline: 5.8177x; 2.5267x over previous
import functools

import jax
import jax.numpy as jnp
from jax import lax
from jax.experimental import pallas as pl
from jax.experimental.pallas import tpu as pltpu
from jax.experimental.pallas import tpu_sc as plsc

_NUM_FIELDS = 26
_FIELD_DIM = 100000
_EMBED_DIM = 16
_BATCH = 16384
_NW = 32
_BPW = _BATCH // _NW
_NVEC = _BPW // 16

_NROW = 2600001
_R = 1024
_NFULL = _NROW // _R
_TAILROW = _NFULL * _R
_TAIL = _NROW - _TAILROW
_CPW_HI = _NFULL - (_NFULL // _NW) * _NW


@functools.partial(
    pl.kernel,
    out_type=jax.ShapeDtypeStruct((_NROW * 16,), jnp.float32),
    mesh=plsc.VectorSubcoreMesh(core_axis_name="c", subcore_axis_name="s"),
    compiler_params=pltpu.CompilerParams(
        use_tc_tiling_on_sc=True, needs_layout_passes=False
    ),
    scratch_types=[
        pltpu.VMEM((16, _R), jnp.float32),
        pltpu.VMEM((16, _R), jnp.float32),
        pltpu.VMEM((_R * 16,), jnp.float32),
        pltpu.VMEM((_R * 16,), jnp.float32),
        pltpu.VMEM((16, _TAIL), jnp.float32),
        pltpu.VMEM((16,), jnp.int32),
        pltpu.SemaphoreType.DMA,
        pltpu.SemaphoreType.DMA,
        pltpu.SemaphoreType.DMA,
        pltpu.SemaphoreType.DMA,
    ],
)
def _linearize_table(tt_hbm, tlin_hbm, in_v0, in_v1, out_v0, out_v1, tin_v,
                     iota_v, is0, is1, os0, os1):
    wid = lax.axis_index("s") * 2 + lax.axis_index("c")
    iota_v[...] = lax.iota(jnp.int32, 16) * 16
    ii16 = iota_v[...]

    nch = jnp.where(wid < _CPW_HI, _NFULL // _NW + 1, _NFULL // _NW)
    in_bufs = (in_v0, in_v1)
    out_bufs = (out_v0, out_v1)
    in_sems = (is0, is1)
    out_sems = (os0, os1)

    def src_at(k):
        return tt_hbm.at[:, pl.ds((wid + _NW * k) * _R, _R)]

    def dst_at(k):
        return tlin_hbm.at[pl.ds((wid + _NW * k) * _R * 16, _R * 16)]

    pltpu.async_copy(src_at(0), in_v0, is0)
    pltpu.async_copy(src_at(1), in_v1, is1)

    def do_pair(kk, carry):
        for b in range(2):
            k = kk * 2 + b

            @pl.when(k < nch)
            def _step(k=k, b=b):
                iv, ov = in_bufs[b], out_bufs[b]
                pltpu.make_async_copy(src_at(k), iv, in_sems[b]).wait()

                @pl.when(k >= 2)
                def _drain_out():
                    pltpu.make_async_copy(ov, dst_at(k - 2), out_sems[b]).wait()

                def tr_vec(g, c2):
                    for c in range(16):
                        plsc.store_scatter(ov, [ii16 + (g * 256 + c)],
                                           iv[c, pl.ds(g * 16, 16)])
                    return c2

                lax.fori_loop(0, _R // 16, tr_vec, 0)
                pltpu.async_copy(ov, dst_at(k), out_sems[b])

                @pl.when(k + 2 < nch)
                def _next_in():
                    pltpu.async_copy(src_at(k + 2), iv, in_sems[b])

        return carry

    lax.fori_loop(0, (_NFULL // _NW + 2) // 2, do_pair, 0)
    pltpu.make_async_copy(out_v0, dst_at(0), os0).wait()
    pltpu.make_async_copy(out_v1, dst_at(1), os1).wait()

    @pl.when(wid == _NW - 1)
    def _tail():
        pltpu.sync_copy(tt_hbm.at[:, pl.ds(_TAILROW, _TAIL)], tin_v)
        starts = [g * 16 for g in range(_TAIL // 16)] + [_TAIL - 16]
        for s in starts:
            for c in range(16):
                plsc.store_scatter(out_v0, [ii16 + (s * 16 + c)],
                                   tin_v[c, pl.ds(s, 16)])
        pltpu.sync_copy(out_v0.at[pl.ds(0, _TAIL * 16)],
                        tlin_hbm.at[pl.ds(_TAILROW * 16, _TAIL * 16)])


@functools.partial(
    pl.kernel,
    out_type=jax.ShapeDtypeStruct((_NUM_FIELDS * _BATCH,), jnp.int32),
    mesh=plsc.VectorSubcoreMesh(core_axis_name="c", subcore_axis_name="s"),
    compiler_params=pltpu.CompilerParams(
        use_tc_tiling_on_sc=True, needs_layout_passes=False
    ),
    scratch_types=[
        pltpu.VMEM((_BPW,), jnp.int32),
        pltpu.VMEM((_BPW,), jnp.int32),
    ],
)
def _make_indices(xt_hbm, idx_hbm, x_v, idx_v):
    wid = lax.axis_index("s") * 2 + lax.axis_index("c")
    b0 = wid * _BPW

    def do_field(f, carry):
        pltpu.sync_copy(xt_hbm.at[f].at[pl.ds(b0, _BPW)], x_v)
        off = f * _FIELD_DIM

        def compute(vi, c2):
            p = pl.ds(vi * 16, 16)
            xv = x_v[p]
            idx_v[p] = jnp.where(xv == 0, 0, xv + off)
            return c2

        lax.fori_loop(0, _NVEC, compute, 0)
        pltpu.sync_copy(idx_v, idx_hbm.at[pl.ds(f * _BATCH + b0, _BPW)])
        return carry

    lax.fori_loop(0, _NUM_FIELDS, do_field, 0)


@functools.partial(
    pl.kernel,
    out_type=jax.ShapeDtypeStruct((_NUM_FIELDS, _EMBED_DIM, _BATCH), jnp.float32),
    mesh=plsc.VectorSubcoreMesh(core_axis_name="c", subcore_axis_name="s"),
    compiler_params=pltpu.CompilerParams(
        use_tc_tiling_on_sc=False, needs_layout_passes=False
    ),
    scratch_types=[
        pltpu.VMEM((_BPW,), jnp.int32),
        pltpu.VMEM((_BPW, _EMBED_DIM), jnp.float32),
        pltpu.VMEM((_EMBED_DIM, _BPW), jnp.float32),
        pltpu.VMEM((16,), jnp.int32),
        pltpu.SemaphoreType.DMA,
    ],
)
def _emb_lookup(idx_hbm, table_hbm, out_hbm, idx_v, rows_v, valt_v, iota_v, sem):
    wid = lax.axis_index("s") * 2 + lax.axis_index("c")
    b0 = wid * _BPW
    iota_v[...] = lax.iota(jnp.int32, 16)

    def do_field(f, carry):
        pltpu.sync_copy(idx_hbm.at[pl.ds(f * _BATCH + b0, _BPW)], idx_v)

        copies = [
            pltpu.async_copy(
                table_hbm.at[idx_v.at[pl.ds(j * 128, 128)]],
                rows_v.at[pl.ds(j * 128, 128)],
                sem,
            )
            for j in range(_BPW // 128)
        ]
        for cp in copies:
            cp.wait()

        def transpose16(g, c2):
            ii = iota_v[...]
            for r16 in range(16):
                r = g * 16 + r16
                rv = jnp.full((16,), r, jnp.int32)
                plsc.store_scatter(valt_v, [ii, rv], rows_v[r, :])
            return c2

        lax.fori_loop(0, _NVEC, transpose16, 0)
        pltpu.sync_copy(valt_v, out_hbm.at[f].at[:, pl.ds(b0, _BPW)])
        return carry

    lax.fori_loop(0, _NUM_FIELDS, do_field, 0)


def kernel(x, table):
    tlin = _linearize_table(table.T)
    t2d = tlin.reshape(_NROW, _EMBED_DIM)
    idx = _make_indices(x.T)
    out = _emb_lookup(idx, t2d)
    return jnp.transpose(out, (2, 0, 1))
